# baseline (speedup 1.0000x reference)
import jax
import jax.numpy as jnp
from jax import lax
from jax.experimental import pallas as pl


_ROWS, _COLS = 128, 32768
_BLOCK_ROWS = 128


def _onehot_argmax_kernel(x_ref, o_ref):
    m = jnp.max(x_ref[...], axis=-1, keepdims=True)
    iota = lax.broadcasted_iota(jnp.int32, (_BLOCK_ROWS, _COLS), 1)
    big = jnp.int32(2**30)
    first = jnp.min(
        jnp.where(x_ref[...] == m, iota, big), axis=-1, keepdims=True
    )
    o_ref[...] = (iota == first).astype(jnp.float32)


def kernel(x):
    grid = (_ROWS // _BLOCK_ROWS,)
    return pl.pallas_call(
        _onehot_argmax_kernel,
        out_shape=jax.ShapeDtypeStruct((_ROWS, _COLS), jnp.float32),
        grid=grid,
        in_specs=[pl.BlockSpec((_BLOCK_ROWS, _COLS), lambda i: (i, 0))],
        out_specs=pl.BlockSpec((_BLOCK_ROWS, _COLS), lambda i: (i, 0)),
    )(x)


# speedup vs baseline: 1.3251x; 1.3251x over previous
import jax
import jax.numpy as jnp
from jax import lax
from jax.experimental import pallas as pl


_ROWS, _COLS = 128, 32768
_BLOCK_ROWS = 64


def _onehot_argmax_kernel(x_ref, o_ref):
    m = jnp.max(x_ref[...], axis=-1, keepdims=True)
    iota = lax.broadcasted_iota(jnp.int32, (_BLOCK_ROWS, _COLS), 1)
    big = jnp.int32(2**30)
    first = jnp.min(
        jnp.where(x_ref[...] == m, iota, big), axis=-1, keepdims=True
    )
    o_ref[...] = (iota == first).astype(jnp.float32)


def kernel(x):
    grid = (_ROWS // _BLOCK_ROWS,)
    return pl.pallas_call(
        _onehot_argmax_kernel,
        out_shape=jax.ShapeDtypeStruct((_ROWS, _COLS), jnp.float32),
        grid=grid,
        in_specs=[pl.BlockSpec((_BLOCK_ROWS, _COLS), lambda i: (i, 0))],
        out_specs=pl.BlockSpec((_BLOCK_ROWS, _COLS), lambda i: (i, 0)),
    )(x)
